# Initial kernel scaffold; baseline (speedup 1.0000x reference)
import jax
import jax.numpy as jnp
from jax.experimental import pallas as pl


def kernel(input_ids, token_type_ids, word_emb, pos_emb, type_emb, gamma, beta):
    raise NotImplementedError("write your pallas kernel here")


# speedup vs baseline: 3.1477x; 3.1477x over previous
import functools

import jax
import jax.numpy as jnp
from jax import lax
from jax.experimental import pallas as pl
from jax.experimental.pallas import tpu as pltpu
from jax.experimental.pallas import tpu_sc as plsc

VOCAB = 100000
HID = 128
B = 1024
S = 200
EPS = 1e-12

NTOK = B * S
NW = 32
TOK_PER_W = NTOK // NW
CHUNK = 128
NCHUNK = TOK_PER_W // CHUNK


def _gather_kernel(ids_hbm, table_hbm, out_hbm, idx_v, buf0, buf1, sem0, sem1):
    wid = lax.axis_index("s") * 2 + lax.axis_index("c")
    base = wid * TOK_PER_W
    pltpu.sync_copy(ids_hbm.at[wid], idx_v)
    bufs = (buf0, buf1)
    sems = (sem0, sem1)

    def start(c):
        return pltpu.async_copy(table_hbm.at[idx_v.at[c]], bufs[c % 2], sems[c % 2])

    handles = [None] * NCHUNK
    handles[0] = start(0)
    for c in range(NCHUNK):
        if c + 1 < NCHUNK:
            handles[c + 1] = start(c + 1)
        handles[c].wait()
        pltpu.sync_copy(bufs[c % 2], out_hbm.at[pl.ds(base + c * CHUNK, CHUNK)])


def _sc_gather(ids, word_emb):
    mesh = plsc.VectorSubcoreMesh(core_axis_name="c", subcore_axis_name="s")
    k = functools.partial(
        pl.kernel,
        mesh=mesh,
        out_type=jax.ShapeDtypeStruct((NTOK, HID), jnp.float32),
        scratch_types=[
            pltpu.VMEM((NCHUNK, CHUNK), jnp.int32),
            pltpu.VMEM((CHUNK, HID), jnp.float32),
            pltpu.VMEM((CHUNK, HID), jnp.float32),
            pltpu.SemaphoreType.DMA,
            pltpu.SemaphoreType.DMA,
        ],
    )(_gather_kernel)
    return k(ids, word_emb)


def _ln_kernel(words_ref, tt_ref, pos_ref, type_ref, gamma_ref, beta_ref, out_ref):
    words = words_ref[0]
    tt = tt_ref[0, 0, :].astype(jnp.float32)
    t0 = type_ref[0]
    t1 = type_ref[1]
    emb = words + pos_ref[...] + t0[None, :] + tt[:, None] * (t1 - t0)[None, :]
    mu = jnp.mean(emb, axis=-1, keepdims=True)
    xc = emb - mu
    var = jnp.mean(xc * xc, axis=-1, keepdims=True)
    y = xc * lax.rsqrt(var + EPS)
    out_ref[0] = y * gamma_ref[0][None, :] + beta_ref[0][None, :]


def _tc_add_ln(words, token_type_ids, pos_emb, type_emb, gamma, beta):
    return pl.pallas_call(
        _ln_kernel,
        grid=(B,),
        in_specs=[
            pl.BlockSpec((1, S, HID), lambda i: (i, 0, 0)),
            pl.BlockSpec((1, 1, S), lambda i: (i, 0, 0)),
            pl.BlockSpec((S, HID), lambda i: (0, 0)),
            pl.BlockSpec((2, HID), lambda i: (0, 0)),
            pl.BlockSpec((1, HID), lambda i: (0, 0)),
            pl.BlockSpec((1, HID), lambda i: (0, 0)),
        ],
        out_specs=pl.BlockSpec((1, S, HID), lambda i: (i, 0, 0)),
        out_shape=jax.ShapeDtypeStruct((B, S, HID), jnp.float32),
    )(words, token_type_ids, pos_emb, type_emb, gamma, beta)


def kernel(input_ids, token_type_ids, word_emb, pos_emb, type_emb, gamma, beta):
    ids = input_ids.astype(jnp.int32).reshape(NW, NCHUNK, CHUNK)
    rows = _sc_gather(ids, word_emb)
    words = rows.reshape(B, S, HID)
    tt3 = token_type_ids.astype(jnp.int32).reshape(B, 1, S)
    pos = pos_emb[:S]
    g2 = gamma.reshape(1, HID)
    b2 = beta.reshape(1, HID)
    return _tc_add_ln(words, tt3, pos, type_emb, g2, b2)


# speedup vs baseline: 10.9253x; 3.4709x over previous
import functools

import jax
import jax.numpy as jnp
from jax import lax
from jax.experimental import pallas as pl
from jax.experimental.pallas import tpu as pltpu
from jax.experimental.pallas import tpu_sc as plsc

VOCAB = 100000
HID = 128
B = 1024
S = 200
EPS = 1e-12

NTOK = B * S
NW = 32
TOK_PER_W = NTOK // NW
CHUNK = 128
NCHUNK = TOK_PER_W // CHUNK


def _gather_kernel(ids_hbm, table_hbm, out_hbm, idx_v, buf0, buf1, sem0, sem1):
    wid = lax.axis_index("s") * 2 + lax.axis_index("c")
    base = wid * TOK_PER_W
    pltpu.sync_copy(ids_hbm.at[wid], idx_v)
    bufs = (buf0, buf1)
    sems = (sem0, sem1)

    def start(c):
        return pltpu.async_copy(table_hbm.at[idx_v.at[c]], bufs[c % 2], sems[c % 2])

    handles = [None] * NCHUNK
    handles[0] = start(0)
    for c in range(NCHUNK):
        if c + 1 < NCHUNK:
            handles[c + 1] = start(c + 1)
        handles[c].wait()
        pltpu.sync_copy(bufs[c % 2], out_hbm.at[pl.ds(base + c * CHUNK, CHUNK)])


def _sc_gather(ids, word_emb):
    mesh = plsc.VectorSubcoreMesh(core_axis_name="c", subcore_axis_name="s")
    k = functools.partial(
        pl.kernel,
        mesh=mesh,
        out_type=jax.ShapeDtypeStruct((NTOK, HID), jnp.float32),
        scratch_types=[
            pltpu.VMEM((NCHUNK, CHUNK), jnp.int32),
            pltpu.VMEM((CHUNK, HID), jnp.float32),
            pltpu.VMEM((CHUNK, HID), jnp.float32),
            pltpu.SemaphoreType.DMA,
            pltpu.SemaphoreType.DMA,
        ],
    )(_gather_kernel)
    return k(ids, word_emb)


BB = 16


def _ln_kernel(words_ref, tt_ref, pos_ref, type_ref, gamma_ref, beta_ref, out_ref):
    words = words_ref[...]
    tt = tt_ref[:, 0, :].astype(jnp.float32)
    t0 = type_ref[0]
    t1 = type_ref[1]
    emb = (words + pos_ref[...][None, :, :] + t0[None, None, :]
           + tt[:, :, None] * (t1 - t0)[None, None, :])
    mu = jnp.mean(emb, axis=-1, keepdims=True)
    xc = emb - mu
    var = jnp.mean(xc * xc, axis=-1, keepdims=True)
    y = xc * lax.rsqrt(var + EPS)
    out_ref[...] = y * gamma_ref[0][None, None, :] + beta_ref[0][None, None, :]


def _tc_add_ln(words, token_type_ids, pos_emb, type_emb, gamma, beta):
    return pl.pallas_call(
        _ln_kernel,
        grid=(B // BB,),
        in_specs=[
            pl.BlockSpec((BB, S, HID), lambda i: (i, 0, 0)),
            pl.BlockSpec((BB, 1, S), lambda i: (i, 0, 0)),
            pl.BlockSpec((S, HID), lambda i: (0, 0)),
            pl.BlockSpec((2, HID), lambda i: (0, 0)),
            pl.BlockSpec((1, HID), lambda i: (0, 0)),
            pl.BlockSpec((1, HID), lambda i: (0, 0)),
        ],
        out_specs=pl.BlockSpec((BB, S, HID), lambda i: (i, 0, 0)),
        out_shape=jax.ShapeDtypeStruct((B, S, HID), jnp.float32),
    )(words, token_type_ids, pos_emb, type_emb, gamma, beta)


def kernel(input_ids, token_type_ids, word_emb, pos_emb, type_emb, gamma, beta):
    ids = input_ids.astype(jnp.int32).reshape(NW, NCHUNK, CHUNK)
    rows = _sc_gather(ids, word_emb)
    words = rows.reshape(B, S, HID)
    tt3 = token_type_ids.astype(jnp.int32).reshape(B, 1, S)
    pos = pos_emb[:S]
    g2 = gamma.reshape(1, HID)
    b2 = beta.reshape(1, HID)
    return _tc_add_ln(words, tt3, pos, type_emb, g2, b2)


# speedup vs baseline: 11.9474x; 1.0936x over previous
import functools

import jax
import jax.numpy as jnp
from jax import lax
from jax.experimental import pallas as pl
from jax.experimental.pallas import tpu as pltpu
from jax.experimental.pallas import tpu_sc as plsc

VOCAB = 100000
HID = 128
B = 1024
S = 200
EPS = 1e-12

NW = 32
NSLAB = 4
SB = B // NSLAB
STOK = SB * S
TOK_PER_W = STOK // NW
CHUNK = 64
NCHUNK = TOK_PER_W // CHUNK
BB = 16
SLAB_STEPS = SB // BB


def _gather_kernel(ids_hbm, table_hbm, out_hbm, idx_v, buf0, buf1, sem0, sem1):
    wid = lax.axis_index("s") * 2 + lax.axis_index("c")
    base = wid * TOK_PER_W
    pltpu.sync_copy(ids_hbm.at[wid], idx_v)
    bufs = (buf0, buf1)
    sems = (sem0, sem1)

    def start(c):
        return pltpu.async_copy(table_hbm.at[idx_v.at[c]], bufs[c % 2], sems[c % 2])

    handles = [None] * NCHUNK
    handles[0] = start(0)
    for c in range(NCHUNK):
        if c + 1 < NCHUNK:
            handles[c + 1] = start(c + 1)
        handles[c].wait()
        pltpu.sync_copy(bufs[c % 2], out_hbm.at[pl.ds(base + c * CHUNK, CHUNK)])


def _sc_gather(ids, word_emb):
    mesh = plsc.VectorSubcoreMesh(core_axis_name="c", subcore_axis_name="s")
    k = functools.partial(
        pl.kernel,
        mesh=mesh,
        out_type=jax.ShapeDtypeStruct((STOK, HID), jnp.float32),
        scratch_types=[
            pltpu.VMEM((NCHUNK, CHUNK), jnp.int32),
            pltpu.VMEM((CHUNK, HID), jnp.float32),
            pltpu.VMEM((CHUNK, HID), jnp.float32),
            pltpu.SemaphoreType.DMA,
            pltpu.SemaphoreType.DMA,
        ],
    )(_gather_kernel)
    return k(ids, word_emb)


def _ln_kernel(words_ref, tt_ref, pos_ref, type_ref, gamma_ref, beta_ref,
               prev_ref, out_ref):
    del prev_ref
    words = words_ref[...]
    tt = tt_ref[:, 0, :].astype(jnp.float32)
    t0 = type_ref[0]
    t1 = type_ref[1]
    emb = (words + pos_ref[...][None, :, :] + t0[None, None, :]
           + tt[:, :, None] * (t1 - t0)[None, None, :])
    mu = jnp.mean(emb, axis=-1, keepdims=True)
    xc = emb - mu
    var = jnp.mean(xc * xc, axis=-1, keepdims=True)
    y = xc * lax.rsqrt(var + EPS)
    out_ref[...] = y * gamma_ref[0][None, None, :] + beta_ref[0][None, None, :]


def _tc_add_ln(slab, words, token_type_ids, pos_emb, type_emb, gamma, beta, prev):
    in_specs = [
        pl.BlockSpec((BB, S, HID), lambda i: (i, 0, 0)),
        pl.BlockSpec((BB, 1, S), lambda i: (i, 0, 0)),
        pl.BlockSpec((S, HID), lambda i: (0, 0)),
        pl.BlockSpec((2, HID), lambda i: (0, 0)),
        pl.BlockSpec((1, HID), lambda i: (0, 0)),
        pl.BlockSpec((1, HID), lambda i: (0, 0)),
    ]
    args = [words, token_type_ids, pos_emb, type_emb, gamma, beta]
    aliases = {}
    body = _ln_kernel
    if prev is not None:
        in_specs.append(pl.BlockSpec(memory_space=pl.ANY))
        args.append(prev)
        aliases = {6: 0}
    else:
        body = functools.partial(
            lambda *refs: _ln_kernel(*refs[:6], None, refs[6]))
    return pl.pallas_call(
        body,
        grid=(SLAB_STEPS,),
        in_specs=in_specs,
        out_specs=pl.BlockSpec(
            (BB, S, HID), lambda i, _s=slab: (_s * SLAB_STEPS + i, 0, 0)),
        out_shape=jax.ShapeDtypeStruct((B, S, HID), jnp.float32),
        input_output_aliases=aliases,
    )(*args)


def kernel(input_ids, token_type_ids, word_emb, pos_emb, type_emb, gamma, beta):
    ids = input_ids.astype(jnp.int32).reshape(NSLAB, NW, NCHUNK, CHUNK)
    tt = token_type_ids.astype(jnp.int32).reshape(NSLAB, SB, 1, S)
    pos = pos_emb[:S]
    g2 = gamma.reshape(1, HID)
    b2 = beta.reshape(1, HID)
    slab_words = [
        _sc_gather(ids[s], word_emb).reshape(SB, S, HID) for s in range(NSLAB)
    ]
    out = None
    for s in range(NSLAB):
        out = _tc_add_ln(s, slab_words[s], tt[s], pos, type_emb, g2, b2, out)
    return out
